# baseline (speedup 1.0000x reference)
import functools

import jax
import jax.numpy as jnp
from jax import lax
from jax.experimental import pallas as pl
from jax.experimental.pallas import tpu as pltpu
from jax.experimental.pallas import tpu_sc as plsc

_NUM_FIELDS = 26
_VOCAB = 100000
_D = 100
_BATCH = 16384
_B_TOTAL = _BATCH * _NUM_FIELDS
_NC = 2
_NS = 16
_NW = _NC * _NS
_ROWS_PER_W = _B_TOTAL // _NW
_CHUNK = 128
_N_CHUNKS = _ROWS_PER_W // _CHUNK

_mesh = plsc.VectorSubcoreMesh(core_axis_name="c", subcore_axis_name="s")


@functools.partial(
    pl.kernel,
    out_type=jax.ShapeDtypeStruct((_B_TOTAL, _D), jnp.float32),
    mesh=_mesh,
    scratch_types=[
        pltpu.VMEM((_N_CHUNKS, _CHUNK), jnp.int32),
        pltpu.VMEM((2, _CHUNK, _D), jnp.float32),
        pltpu.SemaphoreType.DMA,
        pltpu.SemaphoreType.DMA,
    ],
    compiler_params=pltpu.CompilerParams(use_tc_tiling_on_sc=False),
)
def _sc_gather(table_hbm, idx_hbm, out_hbm, idx_v, rows_v, gsem, ssem):
    wid = lax.axis_index("s") * _NC + lax.axis_index("c")
    base = wid * _ROWS_PER_W
    pltpu.sync_copy(idx_hbm.at[wid], idx_v)

    @pl.loop(0, _N_CHUNKS)
    def _chunk(j):
        pltpu.async_copy(table_hbm.at[idx_v.at[j]], rows_v.at[0], gsem).wait()
        pltpu.sync_copy(rows_v.at[0], out_hbm.at[pl.ds(base + j * _CHUNK, _CHUNK)])


def kernel(x_cat, tables):
    flat_table = tables.reshape(_NUM_FIELDS * _VOCAB, _D)
    offs = jnp.arange(_NUM_FIELDS, dtype=jnp.int32) * _VOCAB
    flat_idx = (x_cat.astype(jnp.int32) + offs[None, :]).reshape(
        _NW, _N_CHUNKS, _CHUNK
    )
    out = _sc_gather(flat_table, flat_idx)
    return out.reshape(_BATCH, _NUM_FIELDS * _D)


# speedup vs baseline: 1.5504x; 1.5504x over previous
import functools

import jax
import jax.numpy as jnp
from jax import lax
from jax.experimental import pallas as pl
from jax.experimental.pallas import tpu as pltpu
from jax.experimental.pallas import tpu_sc as plsc

_NUM_FIELDS = 26
_VOCAB = 100000
_D = 100
_BATCH = 16384
_B_TOTAL = _BATCH * _NUM_FIELDS
_NC = 2
_NS = 16
_NW = _NC * _NS
_ROWS_PER_W = _B_TOTAL // _NW
_CHUNK = 128
_N_CHUNKS = _ROWS_PER_W // _CHUNK

_mesh = plsc.VectorSubcoreMesh(core_axis_name="c", subcore_axis_name="s")


@functools.partial(
    pl.kernel,
    out_type=jax.ShapeDtypeStruct((_B_TOTAL, _D), jnp.float32),
    mesh=_mesh,
    scratch_types=[
        pltpu.VMEM((_N_CHUNKS, _CHUNK), jnp.int32),
        pltpu.VMEM((_CHUNK, _D), jnp.float32),
        pltpu.SemaphoreType.DMA,
    ],
)
def _sc_gather(table_hbm, idx_hbm, out_hbm, idx_v, rows_v, gsem):
    wid = lax.axis_index("s") * _NC + lax.axis_index("c")
    base = wid * _ROWS_PER_W
    pltpu.sync_copy(idx_hbm.at[wid], idx_v)

    @pl.loop(0, _N_CHUNKS)
    def _chunk(j):
        @pl.loop(0, _CHUNK // 16)
        def _vec(k):
            vec = idx_v[j, pl.ds(k * 16, 16)]
            for l in range(16):
                pltpu.async_copy(
                    table_hbm.at[pl.ds(vec[l], 1)],
                    rows_v.at[pl.ds(k * 16 + l, 1)],
                    gsem,
                )

        pltpu.make_async_copy(
            table_hbm.at[pl.ds(0, _CHUNK)], rows_v, gsem
        ).wait()
        pltpu.sync_copy(rows_v, out_hbm.at[pl.ds(base + j * _CHUNK, _CHUNK)])


def kernel(x_cat, tables):
    flat_table = tables.reshape(_NUM_FIELDS * _VOCAB, _D)
    offs = jnp.arange(_NUM_FIELDS, dtype=jnp.int32) * _VOCAB
    flat_idx = (x_cat.astype(jnp.int32) + offs[None, :]).reshape(
        _NW, _N_CHUNKS, _CHUNK
    )
    out = _sc_gather(flat_table, flat_idx)
    return out.reshape(_BATCH, _NUM_FIELDS * _D)
